# baseline (speedup 1.0000x reference)
import jax
import jax.numpy as jnp
from jax import lax
from jax.experimental import pallas as pl
from jax.experimental.pallas import tpu as pltpu
from jax.experimental.pallas import tpu_sc as plsc

B, N, L, D = 1, 512, 1024, 64
DP = D + 1
LP = L + 1
V_MSA = 22
NC, NS = 2, 16
NW = NC * NS
N_PER_W = N // NW
CHUNK = 256
CPL = L // CHUNK
N_CHUNKS = N_PER_W * CPL
IDX_CHUNK = 128


def _body(msa_hbm, idx_hbm, emb_hbm, pe_hbm, peq_hbm, out_hbm,
          tbl, embv, peqv, idxv, stage, pev, msav, obuf,
          sem_g, sem_a, sem_b):
    wid = lax.axis_index("s") * NC + lax.axis_index("c")
    n0 = wid * N_PER_W

    pltpu.sync_copy(idx_hbm.at[0], idxv)
    pltpu.sync_copy(msa_hbm.at[0, pl.ds(n0, N_PER_W)], msav)
    pltpu.sync_copy(emb_hbm, embv)
    pltpu.sync_copy(peq_hbm, peqv)

    peq_regs = [[peqv[s, pl.ds(16 * j, 16)] for j in range(4)] for s in range(2)]
    for s in range(2):
        for i in range(V_MSA):
            for j in range(4):
                tbl[pl.ds((s * V_MSA + i) * DP + 16 * j, 16)] = (
                    embv[i, pl.ds(16 * j, 16)] + peq_regs[s][j])

    col16 = lax.iota(jnp.int32, 16)
    cvecs = [(col16 + 16 * j) * LP for j in range(4)]
    for k in range(L // IDX_CHUNK):
        pltpu.async_copy(
            pe_hbm.at[idxv.at[pl.ds(k * IDX_CHUNK, IDX_CHUNK)]],
            stage, sem_g).wait()

        def repack(r, _, *, k=k):
            lpos = k * IDX_CHUNK + r
            for j in range(4):
                plsc.store_scatter(pev, [cvecs[j] + lpos],
                                   stage[r, pl.ds(16 * j, 16)])
            return 0

        lax.fori_loop(0, IDX_CHUNK, repack, 0)
    sems = [sem_a, sem_b]

    def chunk_do(cc, b):
        n_rel = cc // CPL
        l0 = (cc % CPL) * CHUNK
        ng = n0 + n_rel
        off = jnp.where(ng == 0, 0, V_MSA).astype(jnp.int32)

        def lblock(lb, _):
            lbase = l0 + lb * 16
            m16 = msav[n_rel, pl.ds(lbase, 16)]
            tvec = (m16 + off) * DP
            for d0 in range(0, D, 16):
                gs = [plsc.load_gather(tbl, [tvec + (d0 + i)])
                      for i in range(16)]
                ps = [pev[pl.ds((d0 + i) * LP + lbase, 16)]
                      for i in range(16)]
                for i in range(16):
                    obuf[b, d0 + i, pl.ds(lb * 16, 16)] = gs[i] + ps[i]
            return 0

        lax.fori_loop(0, CHUNK // 16, lblock, 0)
        pltpu.async_copy(obuf.at[b], out_hbm.at[ng, :, pl.ds(l0, CHUNK)],
                         sems[b])

    def drain(b):
        pltpu.make_async_copy(obuf.at[b],
                              out_hbm.at[0, :, pl.ds(0, CHUNK)],
                              sems[b]).wait()

    chunk_do(jnp.int32(0), 0)
    chunk_do(jnp.int32(1), 1)

    def outer(co, _):
        for b in range(2):
            drain(b)
            chunk_do(co * 2 + b, b)
        return 0

    lax.fori_loop(1, N_CHUNKS // 2, outer, 0)
    drain(0)
    drain(1)


@jax.jit
def kernel(msa, idx, emb_W, pe_buf, pe_q):
    mesh = plsc.VectorSubcoreMesh(core_axis_name="c", subcore_axis_name="s",
                                  num_cores=NC, num_subcores=NS)
    fn = pl.kernel(
        _body,
        out_type=jax.ShapeDtypeStruct((N, D, L), jnp.float32),
        mesh=mesh,
        scratch_types=[
            pltpu.VMEM((2 * V_MSA * DP,), jnp.float32),
            pltpu.VMEM((V_MSA, D), jnp.float32),
            pltpu.VMEM((2, D), jnp.float32),
            pltpu.VMEM((L,), jnp.int32),
            pltpu.VMEM((IDX_CHUNK, D), jnp.float32),
            pltpu.VMEM((D * LP,), jnp.float32),
            pltpu.VMEM((N_PER_W, L), jnp.int32),
            pltpu.VMEM((2, D, CHUNK), jnp.float32),
            pltpu.SemaphoreType.DMA,
            pltpu.SemaphoreType.DMA,
            pltpu.SemaphoreType.DMA,
        ],
        compiler_params=pltpu.CompilerParams(needs_layout_passes=False,
                                             use_tc_tiling_on_sc=False),
    )
    out_ndl = fn(msa, idx, emb_W, pe_buf, pe_q)
    return jnp.swapaxes(out_ndl, 1, 2)[None]


# speedup vs baseline: 1.1175x; 1.1175x over previous
import jax
import jax.numpy as jnp
from jax import lax
from jax.experimental import pallas as pl
from jax.experimental.pallas import tpu as pltpu
from jax.experimental.pallas import tpu_sc as plsc

B, N, L, D = 1, 512, 1024, 64
DP = D + 1
V_MSA = 22
NC, NS = 2, 16
NW = NC * NS
N_PER_W = N // NW
CHUNK = 256
CPL = L // CHUNK
N_CHUNKS = N_PER_W * CPL
L_PER_S = L // NS
NBUF = 3


def _body(msa_hbm, idx_hbm, emb_hbm, pe_hbm, peq_hbm, out_hbm,
          tbl, embv, peqv, idxv, stage, part, msav, obuf, pe_sh,
          sem_g, sem_p, sem_o0, sem_o1, sem_o2, sem_i0, sem_i1, sem_i2):
    sid = lax.axis_index("s")
    wid = sid * NC + lax.axis_index("c")
    n0 = wid * N_PER_W
    ls = sid * L_PER_S

    pltpu.sync_copy(idx_hbm.at[0], idxv)
    gat = pltpu.async_copy(
        pe_hbm.at[idxv.at[pl.ds(ls, L_PER_S)]], stage, sem_g)

    pltpu.sync_copy(msa_hbm.at[0, pl.ds(n0, N_PER_W)], msav)
    pltpu.sync_copy(emb_hbm, embv)
    pltpu.sync_copy(peq_hbm, peqv)

    peq_regs = [[peqv[s, pl.ds(16 * j, 16)] for j in range(4)] for s in range(2)]
    for s in range(2):
        for i in range(V_MSA):
            for j in range(4):
                tbl[pl.ds((s * V_MSA + i) * DP + 16 * j, 16)] = (
                    embv[i, pl.ds(16 * j, 16)] + peq_regs[s][j])

    col16 = lax.iota(jnp.int32, 16)
    cvecs = [(col16 + 16 * j) * L_PER_S for j in range(4)]
    gat.wait()

    def transpose_row(r, _):
        for j in range(4):
            plsc.store_scatter(part, [cvecs[j] + r], stage[r, pl.ds(16 * j, 16)])
        return 0

    lax.fori_loop(0, L_PER_S, transpose_row, 0)

    lq = ls // CHUNK
    lr = ls - lq * CHUNK
    pubs = [pltpu.async_copy(part.at[pl.ds(d * L_PER_S, L_PER_S)],
                             pe_sh.at[lq, d, pl.ds(lr, L_PER_S)], sem_p)
            for d in range(D)]
    for p in pubs:
        p.wait()
    plsc.subcore_barrier()

    sem_o = [sem_o0, sem_o1, sem_o2]
    sem_i = [sem_i0, sem_i1, sem_i2]

    def issue_init(cc, b):
        pltpu.async_copy(pe_sh.at[cc % CPL], obuf.at[b], sem_i[b])

    def wait_init(b):
        pltpu.make_async_copy(pe_sh.at[0], obuf.at[b], sem_i[b]).wait()

    def issue_out(cc, b):
        ng = n0 + cc // CPL
        l0 = (cc % CPL) * CHUNK
        pltpu.async_copy(obuf.at[b], out_hbm.at[ng, :, pl.ds(l0, CHUNK)],
                         sem_o[b])

    def drain_out(b):
        pltpu.make_async_copy(obuf.at[b], out_hbm.at[0, :, pl.ds(0, CHUNK)],
                              sem_o[b]).wait()

    def compute(cc, b):
        n_rel = cc // CPL
        l0 = (cc % CPL) * CHUNK
        off = jnp.where(n0 + n_rel == 0, 0, V_MSA).astype(jnp.int32)

        def lblock(lb, _):
            lbase = l0 + lb * 16
            m16 = msav[n_rel, pl.ds(lbase, 16)]
            tvec = (m16 + off) * DP
            for d0 in range(0, D, 16):
                gs = [plsc.load_gather(tbl, [tvec + (d0 + i)])
                      for i in range(16)]
                for i in range(16):
                    plsc.addupdate(obuf.at[b, d0 + i, pl.ds(lb * 16, 16)],
                                   gs[i])
            return 0

        lax.fori_loop(0, CHUNK // 16, lblock, 0)

    issue_init(jnp.int32(0), 0)
    wait_init(0)
    issue_init(jnp.int32(1), 1)
    compute(jnp.int32(0), 0)
    issue_out(jnp.int32(0), 0)
    wait_init(1)
    issue_init(jnp.int32(2), 2)
    compute(jnp.int32(1), 1)
    issue_out(jnp.int32(1), 1)
    wait_init(2)
    drain_out(0)
    issue_init(jnp.int32(3), 0)
    compute(jnp.int32(2), 2)
    issue_out(jnp.int32(2), 2)

    def outer(co, _):
        for j in range(NBUF):
            cc = co * NBUF + j
            b = j
            wait_init(b)
            bn = (j + 1) % NBUF
            drain_out(bn)
            issue_init(cc + 1, bn)
            compute(cc, b)
            issue_out(cc, b)
        return 0

    lax.fori_loop(1, N_CHUNKS // NBUF, outer, 0)

    wait_init(0)
    compute(jnp.int32(N_CHUNKS - 1), 0)
    issue_out(jnp.int32(N_CHUNKS - 1), 0)
    drain_out(1)
    drain_out(2)
    drain_out(0)


@jax.jit
def kernel(msa, idx, emb_W, pe_buf, pe_q):
    mesh = plsc.VectorSubcoreMesh(core_axis_name="c", subcore_axis_name="s",
                                  num_cores=NC, num_subcores=NS)
    fn = pl.kernel(
        _body,
        out_type=jax.ShapeDtypeStruct((N, D, L), jnp.float32),
        mesh=mesh,
        scratch_types=[
            pltpu.VMEM((2 * V_MSA * DP,), jnp.float32),
            pltpu.VMEM((V_MSA, D), jnp.float32),
            pltpu.VMEM((2, D), jnp.float32),
            pltpu.VMEM((L,), jnp.int32),
            pltpu.VMEM((L_PER_S, D), jnp.float32),
            pltpu.VMEM((D * L_PER_S,), jnp.float32),
            pltpu.VMEM((N_PER_W, L), jnp.int32),
            pltpu.VMEM((NBUF, D, CHUNK), jnp.float32),
            pltpu.VMEM_SHARED((CPL, D, CHUNK), jnp.float32),
            pltpu.SemaphoreType.DMA,
            pltpu.SemaphoreType.DMA,
            pltpu.SemaphoreType.DMA,
            pltpu.SemaphoreType.DMA,
            pltpu.SemaphoreType.DMA,
            pltpu.SemaphoreType.DMA,
            pltpu.SemaphoreType.DMA,
            pltpu.SemaphoreType.DMA,
        ],
        compiler_params=pltpu.CompilerParams(needs_layout_passes=False,
                                             use_tc_tiling_on_sc=False),
    )
    out_ndl = fn(msa, idx, emb_W, pe_buf, pe_q)
    return jnp.swapaxes(out_ndl, 1, 2)[None]
